# baseline (speedup 1.0000x reference)
import jax
import jax.numpy as jnp
from jax import lax
from jax.experimental import pallas as pl
from jax.experimental.pallas import tpu as pltpu
from jax.experimental.pallas import tpu_sc as plsc

VOCAB = 1000000
D = 64
B = 4096
L = 200
LOUT = L + 2
CLS_ID = 1
PAD_ID = 0

NC, NS = 2, 16
NW = NC * NS
BPW = B // NW
P = 208
NCHUNK = 4
CHUNK = (2 * P) // NCHUNK


def _extract(ubuf, idxbuf):
    iota = lax.iota(jnp.int32, 16)
    zeros = iota * 0
    twos = zeros + 2
    for k in range(P // 16):
        lvec = iota + (16 * k)
        valid = lvec < L
        i0 = plsc.load_gather(ubuf, [lvec, zeros])
        i2 = plsc.load_gather(ubuf, [lvec, twos])
        idxbuf[pl.ds(16 * k, 16)] = jnp.where(valid, i0, 0)
        idxbuf[pl.ds(P + 16 * k, 16)] = jnp.where(valid, i2, 0)


def _start_gathers(table_hbm, idxbuf, rows, sem):
    for j in range(NCHUNK):
        pltpu.async_copy(table_hbm.at[idxbuf.at[pl.ds(j * CHUNK, CHUNK)]],
                         rows.at[pl.ds(j * CHUNK, CHUNK)], sem)


def _drain_gathers(table_hbm, rows, sem):
    pltpu.make_async_copy(table_hbm.at[pl.ds(0, 2 * P)], rows, sem).wait()


def _start_ubuf(unfold_hbm, ubuf, sem, b):
    pltpu.async_copy(unfold_hbm.at[b], ubuf.at[pl.ds(0, L)], sem)


def _drain_ubuf(unfold_hbm, ubuf, sem):
    pltpu.make_async_copy(unfold_hbm.at[0], ubuf.at[pl.ds(0, L)], sem).wait()


def _compute(rows, outb):
    @plsc.parallel_loop(0, L, unroll=4)
    def _(l):
        for j in range(D // 16):
            sl = pl.ds(16 * j, 16)
            outb[1 + l, sl] = rows[l, sl] + rows[P + l, sl]


def _body(unfold_hbm, table_hbm, out_hbm, ubuf0, ubuf1, idxbuf0, idxbuf1,
          rows0, rows1, outb0, outb1, cidx,
          usem0, usem1, gsem0, gsem1, osem0, osem1, csem):
    wid = lax.axis_index("s") * NC + lax.axis_index("c")
    base = wid * BPW

    cidx[...] = jnp.where(lax.iota(jnp.int32, 16) < 1, CLS_ID, PAD_ID)
    pltpu.async_copy(table_hbm.at[cidx], rows0.at[pl.ds(0, 16)], csem).wait()
    for outb in (outb0, outb1):
        for j in range(D // 16):
            sl = pl.ds(16 * j, 16)
            c = rows0[0, sl]
            p = rows0[1, sl]
            outb[0, sl] = c + c
            outb[LOUT - 1, sl] = p + p

    _start_ubuf(unfold_hbm, ubuf0, usem0, base)
    _start_ubuf(unfold_hbm, ubuf1, usem1, base + 1)
    _drain_ubuf(unfold_hbm, ubuf0, usem0)
    _extract(ubuf0, idxbuf0)
    _start_gathers(table_hbm, idxbuf0, rows0, gsem0)

    def _iter(k, carry):
        i0 = 2 * k
        _drain_ubuf(unfold_hbm, ubuf1, usem1)
        _extract(ubuf1, idxbuf1)
        _drain_gathers(table_hbm, rows0, gsem0)
        _start_gathers(table_hbm, idxbuf1, rows1, gsem1)

        @pl.when(k < BPW // 2 - 1)
        def _():
            _start_ubuf(unfold_hbm, ubuf0, usem0, base + i0 + 2)

        @pl.when(k >= 1)
        def _():
            pltpu.make_async_copy(outb0, out_hbm.at[base], osem0).wait()

        _compute(rows0, outb0)
        pltpu.async_copy(outb0, out_hbm.at[base + i0], osem0)

        @pl.when(k < BPW // 2 - 1)
        def _():
            _drain_ubuf(unfold_hbm, ubuf0, usem0)
            _extract(ubuf0, idxbuf0)

        _drain_gathers(table_hbm, rows1, gsem1)

        @pl.when(k < BPW // 2 - 1)
        def _():
            _start_gathers(table_hbm, idxbuf0, rows0, gsem0)

        @pl.when(k < BPW // 2 - 1)
        def _():
            _start_ubuf(unfold_hbm, ubuf1, usem1, base + i0 + 3)

        @pl.when(k >= 1)
        def _():
            pltpu.make_async_copy(outb1, out_hbm.at[base], osem1).wait()

        _compute(rows1, outb1)
        pltpu.async_copy(outb1, out_hbm.at[base + i0 + 1], osem1)
        return carry

    lax.fori_loop(0, BPW // 2, _iter, 0)

    pltpu.make_async_copy(outb0, out_hbm.at[base], osem0).wait()
    pltpu.make_async_copy(outb1, out_hbm.at[base], osem1).wait()


@jax.jit
def kernel(unfold, emb_table):
    mesh = plsc.VectorSubcoreMesh(core_axis_name="c", subcore_axis_name="s",
                                  num_cores=NC, num_subcores=NS)
    run = pl.kernel(
        _body,
        out_type=jax.ShapeDtypeStruct((B, LOUT, D), jnp.float32),
        mesh=mesh,
        compiler_params=pltpu.CompilerParams(use_tc_tiling_on_sc=False,
                                             needs_layout_passes=False),
        scratch_types=[
            pltpu.VMEM((P, 3), jnp.int32),
            pltpu.VMEM((P, 3), jnp.int32),
            pltpu.VMEM((2 * P,), jnp.int32),
            pltpu.VMEM((2 * P,), jnp.int32),
            pltpu.VMEM((2 * P, D), jnp.float32),
            pltpu.VMEM((2 * P, D), jnp.float32),
            pltpu.VMEM((LOUT, D), jnp.float32),
            pltpu.VMEM((LOUT, D), jnp.float32),
            pltpu.VMEM((16,), jnp.int32),
            pltpu.SemaphoreType.DMA,
            pltpu.SemaphoreType.DMA,
            pltpu.SemaphoreType.DMA,
            pltpu.SemaphoreType.DMA,
            pltpu.SemaphoreType.DMA,
            pltpu.SemaphoreType.DMA,
            pltpu.SemaphoreType.DMA,
        ],
    )
    return run(unfold.astype(jnp.int32), emb_table)


# speedup vs baseline: 3.8636x; 3.8636x over previous
import jax
import jax.numpy as jnp
from jax import lax
from jax.experimental import pallas as pl
from jax.experimental.pallas import tpu as pltpu
from jax.experimental.pallas import tpu_sc as plsc

VOCAB = 1000000
D = 64
B = 4096
L = 200
LOUT = L + 2
CLS_ID = 1
PAD_ID = 0

NC, NS = 2, 16
NW = NC * NS
BPW = B // NW
NCHUNK = 4
CHUNK = (2 * L) // NCHUNK


def _start_gathers(table_hbm, idx_all, rows, sem, i):
    for j in range(NCHUNK):
        pltpu.async_copy(table_hbm.at[idx_all.at[i, j]],
                         rows.at[pl.ds(j * CHUNK, CHUNK)], sem)


def _drain_gathers(table_hbm, rows, sem):
    pltpu.make_async_copy(table_hbm.at[pl.ds(0, 2 * L)], rows, sem).wait()


def _compute(rows, outb):
    @plsc.parallel_loop(0, L, unroll=4)
    def _(l):
        for j in range(D // 16):
            sl = pl.ds(16 * j, 16)
            outb[1 + l, sl] = rows[l, sl] + rows[L + l, sl]


def _body(idx_hbm, table_hbm, out_hbm, idx_all, rows0, rows1, outb0, outb1,
          cidx, gsem0, gsem1, osem0, osem1, csem):
    wid = lax.axis_index("s") * NC + lax.axis_index("c")
    base = wid * BPW

    pltpu.sync_copy(idx_hbm.at[wid], idx_all)

    cidx[...] = jnp.where(lax.iota(jnp.int32, 16) < 1, CLS_ID, PAD_ID)
    pltpu.async_copy(table_hbm.at[cidx], rows0.at[pl.ds(0, 16)], csem).wait()
    for outb in (outb0, outb1):
        for j in range(D // 16):
            sl = pl.ds(16 * j, 16)
            c = rows0[0, sl]
            p = rows0[1, sl]
            outb[0, sl] = c + c
            outb[LOUT - 1, sl] = p + p

    _start_gathers(table_hbm, idx_all, rows0, gsem0, 0)

    def _steady(k, g_next0, g_next1, w_out0, w_out1):
        i0 = 2 * k
        _drain_gathers(table_hbm, rows0, gsem0)
        if g_next0:
            _start_gathers(table_hbm, idx_all, rows1, gsem1, i0 + 1)
        if w_out0:
            pltpu.make_async_copy(outb0, out_hbm.at[base], osem0).wait()
        _compute(rows0, outb0)
        pltpu.async_copy(outb0, out_hbm.at[base + i0], osem0)

        _drain_gathers(table_hbm, rows1, gsem1)
        if g_next1:
            _start_gathers(table_hbm, idx_all, rows0, gsem0, i0 + 2)
        if w_out1:
            pltpu.make_async_copy(outb1, out_hbm.at[base], osem1).wait()
        _compute(rows1, outb1)
        pltpu.async_copy(outb1, out_hbm.at[base + i0 + 1], osem1)

    _steady(0, True, True, False, False)

    def _loop_body(k, carry):
        _steady(k, True, True, True, True)
        return carry

    lax.fori_loop(1, BPW // 2 - 1, _loop_body, 0)

    _steady(BPW // 2 - 1, True, False, True, True)

    pltpu.make_async_copy(outb0, out_hbm.at[base], osem0).wait()
    pltpu.make_async_copy(outb1, out_hbm.at[base], osem1).wait()


@jax.jit
def kernel(unfold, emb_table):
    u = unfold.astype(jnp.int32).reshape(B, 3 * L)
    idx = jnp.concatenate([u[:, 0::3], u[:, 2::3]], axis=1)
    idx = idx.reshape(NW, BPW, NCHUNK, CHUNK)
    mesh = plsc.VectorSubcoreMesh(core_axis_name="c", subcore_axis_name="s",
                                  num_cores=NC, num_subcores=NS)
    run = pl.kernel(
        _body,
        out_type=jax.ShapeDtypeStruct((B, LOUT, D), jnp.float32),
        mesh=mesh,
        compiler_params=pltpu.CompilerParams(use_tc_tiling_on_sc=False),
        scratch_types=[
            pltpu.VMEM((BPW, NCHUNK, CHUNK), jnp.int32),
            pltpu.VMEM((2 * L, D), jnp.float32),
            pltpu.VMEM((2 * L, D), jnp.float32),
            pltpu.VMEM((LOUT, D), jnp.float32),
            pltpu.VMEM((LOUT, D), jnp.float32),
            pltpu.VMEM((16,), jnp.int32),
            pltpu.SemaphoreType.DMA,
            pltpu.SemaphoreType.DMA,
            pltpu.SemaphoreType.DMA,
            pltpu.SemaphoreType.DMA,
            pltpu.SemaphoreType.DMA,
        ],
    )
    return run(idx, emb_table)


# speedup vs baseline: 4.1491x; 1.0739x over previous
import jax
import jax.numpy as jnp
from jax import lax
from jax.experimental import pallas as pl
from jax.experimental.pallas import tpu as pltpu
from jax.experimental.pallas import tpu_sc as plsc

VOCAB = 1000000
D = 64
B = 4096
L = 200
LOUT = L + 2
CLS_ID = 1
PAD_ID = 0

NC, NS = 2, 16
NW = NC * NS
BPW = B // NW
NCHUNK = 4
CHUNK = (2 * L) // NCHUNK
PR = LOUT * D // 128


def _start_gathers(table_hbm, idx_all, rows, sem, i):
    for j in range(NCHUNK):
        pltpu.async_copy(table_hbm.at[idx_all.at[i, j]],
                         rows.at[pl.ds(j * CHUNK, CHUNK)], sem)


def _drain_gathers(table_hbm, rows, sem):
    pltpu.make_async_copy(table_hbm.at[pl.ds(0, 2 * L)], rows, sem).wait()


def _compute(rows, outb):
    @plsc.parallel_loop(0, L // 2, unroll=2)
    def _(m):
        for j in range(D // 16):
            sl = pl.ds(16 * j, 16)
            outb[m, pl.ds(64 + 16 * j, 16)] = (
                rows[4 * m, sl] + rows[4 * m + 1, sl])
            outb[m + 1, pl.ds(16 * j, 16)] = (
                rows[4 * m + 2, sl] + rows[4 * m + 3, sl])


def _body(idx_hbm, table_hbm, out_hbm, idx_all, rows0, rows1, outb0, outb1,
          cidx, gsem0, gsem1, osem0, osem1, csem):
    wid = lax.axis_index("s") * NC + lax.axis_index("c")
    base = wid * BPW

    pltpu.sync_copy(idx_hbm.at[wid], idx_all)

    cidx[...] = jnp.where(lax.iota(jnp.int32, 16) < 1, CLS_ID, PAD_ID)
    pltpu.async_copy(table_hbm.at[cidx], rows0.at[pl.ds(0, 16)], csem).wait()
    for outb in (outb0, outb1):
        for j in range(D // 16):
            sl = pl.ds(16 * j, 16)
            c = rows0[0, sl]
            p = rows0[1, sl]
            outb[0, sl] = c + c
            outb[PR - 1, pl.ds(64 + 16 * j, 16)] = p + p

    _start_gathers(table_hbm, idx_all, rows0, gsem0, 0)

    def _steady(k, g_next0, g_next1, w_out0, w_out1):
        i0 = 2 * k
        _drain_gathers(table_hbm, rows0, gsem0)
        if g_next0:
            _start_gathers(table_hbm, idx_all, rows1, gsem1, i0 + 1)
        if w_out0:
            pltpu.make_async_copy(
                outb0, out_hbm.at[pl.ds(base * PR, PR)], osem0).wait()
        _compute(rows0, outb0)
        pltpu.async_copy(
            outb0, out_hbm.at[pl.ds((base + i0) * PR, PR)], osem0)

        _drain_gathers(table_hbm, rows1, gsem1)
        if g_next1:
            _start_gathers(table_hbm, idx_all, rows0, gsem0, i0 + 2)
        if w_out1:
            pltpu.make_async_copy(
                outb1, out_hbm.at[pl.ds(base * PR, PR)], osem1).wait()
        _compute(rows1, outb1)
        pltpu.async_copy(
            outb1, out_hbm.at[pl.ds((base + i0 + 1) * PR, PR)], osem1)

    _steady(0, True, True, False, False)

    def _loop_body(k, carry):
        _steady(k, True, True, True, True)
        return carry

    lax.fori_loop(1, BPW // 2 - 1, _loop_body, 0)

    _steady(BPW // 2 - 1, True, False, True, True)

    pltpu.make_async_copy(outb0, out_hbm.at[pl.ds(base * PR, PR)],
                          osem0).wait()
    pltpu.make_async_copy(outb1, out_hbm.at[pl.ds(base * PR, PR)],
                          osem1).wait()


@jax.jit
def kernel(unfold, emb_table):
    idx = unfold.astype(jnp.int32)[:, :, 0::2]
    idx = idx.reshape(NW, BPW, NCHUNK, CHUNK)
    mesh = plsc.VectorSubcoreMesh(core_axis_name="c", subcore_axis_name="s",
                                  num_cores=NC, num_subcores=NS)
    run = pl.kernel(
        _body,
        out_type=jax.ShapeDtypeStruct((B * PR, 128), jnp.float32),
        mesh=mesh,
        compiler_params=pltpu.CompilerParams(use_tc_tiling_on_sc=False),
        scratch_types=[
            pltpu.VMEM((BPW, NCHUNK, CHUNK), jnp.int32),
            pltpu.VMEM((2 * L, D), jnp.float32),
            pltpu.VMEM((2 * L, D), jnp.float32),
            pltpu.VMEM((PR, 128), jnp.float32),
            pltpu.VMEM((PR, 128), jnp.float32),
            pltpu.VMEM((16,), jnp.int32),
            pltpu.SemaphoreType.DMA,
            pltpu.SemaphoreType.DMA,
            pltpu.SemaphoreType.DMA,
            pltpu.SemaphoreType.DMA,
            pltpu.SemaphoreType.DMA,
        ],
    )
    return run(idx, emb_table).reshape(B, LOUT, D)
